# baseline (speedup 1.0000x reference)
import functools

import jax
import jax.numpy as jnp
from jax import lax
from jax.experimental import pallas as pl
from jax.experimental.pallas import tpu as pltpu
from jax.experimental.pallas import tpu_sc as plsc

_NC = 2
_NS = 16
_LANES = 16
_NW = _NC * _NS

_CHUNK = 128


@functools.lru_cache(maxsize=None)
def _build_gather(n_rows: int, d: int, table_rows: int):
    elems = n_rows * d
    assert elems % (_NW * _CHUNK) == 0
    per_w = elems // _NW
    n_ch = per_w // _CHUNK
    assert d % _LANES == 0 and _CHUNK % d == 0
    vregs_per_chunk = _CHUNK // _LANES

    mesh = plsc.VectorSubcoreMesh(core_axis_name="c", subcore_axis_name="s")

    @functools.partial(
        pl.kernel,
        mesh=mesh,
        out_type=jax.ShapeDtypeStruct((_NW, n_ch, _CHUNK), jnp.float32),
        scratch_types=[
            pltpu.VMEM((n_ch, _CHUNK), jnp.int32),
            pltpu.VMEM((n_ch, _CHUNK), jnp.float32),
            pltpu.SemaphoreType.DMA,
        ],
    )
    def gather_kernel(table_hbm, idx_hbm, out_hbm, idx_v, val_v, sem):
        wid = lax.axis_index("s") * _NC + lax.axis_index("c")

        pltpu.sync_copy(idx_hbm.at[wid], idx_v)

        lane = lax.broadcasted_iota(jnp.int32, (_LANES,), 0)

        def compute(c, carry):
            for v in range(vregs_per_chunk):
                sl = pl.ds(v * _LANES, _LANES)
                col0 = (v * _LANES) % d
                idx_v[c, sl] = idx_v[c, sl] * d + (lane + col0)
            return carry

        lax.fori_loop(0, n_ch, compute, 0)

        def fire(c, carry):
            pltpu.async_copy(table_hbm.at[idx_v.at[c]], val_v.at[c], sem)
            return carry

        lax.fori_loop(0, n_ch, fire, 0)

        pltpu.make_async_copy(out_hbm.at[wid], val_v, sem).wait()

        pltpu.sync_copy(val_v, out_hbm.at[wid])

    return gather_kernel


def kernel(input, index, _):
    table_rows, d = input.shape
    n_rows = index.shape[0]
    table_flat = input.reshape(table_rows * d)
    per_w = (n_rows * d) // _NW
    idx3 = index.reshape(_NW, per_w // _CHUNK, _CHUNK)
    gathered = _build_gather(n_rows, d, table_rows)(table_flat, idx3)
    return (input, index, gathered.reshape(n_rows, d))


# speedup vs baseline: 1.6455x; 1.6455x over previous
import functools

import jax
import jax.numpy as jnp
from jax import lax
from jax.experimental import pallas as pl
from jax.experimental.pallas import tpu as pltpu
from jax.experimental.pallas import tpu_sc as plsc

_NC = 2
_NS = 16
_LANES = 16
_NW = _NC * _NS

_CHUNK = 128
_RBLK = 131072


@functools.lru_cache(maxsize=None)
def _build_format(d: int, table_rows: int):
    nj = pl.cdiv(table_rows, _RBLK)
    ni = d // 8

    def fmt_kernel(in_ref, pass_ref, flat_ref):
        x = in_ref[...]
        pass_ref[...] = x
        flat_ref[...] = x.reshape(1, 8, _RBLK)

    return pl.pallas_call(
        fmt_kernel,
        grid=(ni, nj),
        in_specs=[pl.BlockSpec((8, _RBLK), lambda i, j: (i, j))],
        out_specs=[
            pl.BlockSpec((8, _RBLK), lambda i, j: (i, j)),
            pl.BlockSpec((1, 8, _RBLK), lambda i, j: (i * nj + j, 0, 0)),
        ],
        out_shape=[
            jax.ShapeDtypeStruct((d, table_rows), jnp.float32),
            jax.ShapeDtypeStruct((ni * nj, 8, _RBLK), jnp.float32),
        ],
        compiler_params=pltpu.CompilerParams(
            dimension_semantics=("parallel", "arbitrary"),
        ),
    )


@functools.lru_cache(maxsize=None)
def _build_gather(n_rows: int, d: int, table_rows: int):
    assert d % _NW == 0
    cols_per_w = d // _NW
    assert n_rows % _CHUNK == 0
    n_ch = n_rows // _CHUNK
    vregs_per_chunk = _CHUNK // _LANES
    nj = pl.cdiv(table_rows, _RBLK)

    mesh = plsc.VectorSubcoreMesh(core_axis_name="c", subcore_axis_name="s")

    @functools.partial(
        pl.kernel,
        mesh=mesh,
        out_type=jax.ShapeDtypeStruct((d, n_ch, _CHUNK), jnp.float32),
        scratch_types=[
            pltpu.VMEM((n_ch, _CHUNK), jnp.int32),
            pltpu.VMEM((n_ch, _CHUNK), jnp.float32),
            pltpu.SemaphoreType.DMA,
        ],
    )
    def gather_kernel(flat_hbm, idxT_hbm, outT_hbm, idx_v, val_v, sem):
        wid = lax.axis_index("s") * _NC + lax.axis_index("c")

        def do_col(k, carry):
            j = wid * cols_per_w + k

            pltpu.sync_copy(idxT_hbm.at[j], idx_v)

            base = ((j // 8) * nj) * (8 * _RBLK) + (j % 8) * _RBLK

            def conv(c, carry2):
                for v in range(vregs_per_chunk):
                    sl = pl.ds(v * _LANES, _LANES)
                    x = idx_v[c, sl]
                    f = base + ((x >> 17) << 20)
                    idx_v[c, sl] = f + (x & (_RBLK - 1))
                return carry2

            lax.fori_loop(0, n_ch, conv, 0)

            def fire(c, carry2):
                pltpu.async_copy(flat_hbm.at[idx_v.at[c]], val_v.at[c], sem)
                return carry2

            lax.fori_loop(0, n_ch, fire, 0)
            pltpu.make_async_copy(outT_hbm.at[j], val_v, sem).wait()

            pltpu.sync_copy(val_v, outT_hbm.at[j])
            return carry

        lax.fori_loop(0, cols_per_w, do_col, 0)

    return gather_kernel


def kernel(input, index, _):
    table_rows, d = input.shape
    n_rows = index.shape[0]

    passT, flat3 = _build_format(d, table_rows)(input.T)
    flat = flat3.reshape(flat3.shape[0] * 8 * _RBLK)

    idxT3 = index.T.reshape(d, n_rows // _CHUNK, _CHUNK)
    gathered = _build_gather(n_rows, d, table_rows)(flat, idxT3)
    return (passT.T, index, gathered.reshape(d, n_rows).T)


# speedup vs baseline: 2.6289x; 1.5976x over previous
import functools

import jax
import jax.numpy as jnp
from jax import lax
from jax.experimental import pallas as pl
from jax.experimental.pallas import tpu as pltpu
from jax.experimental.pallas import tpu_sc as plsc

_NC = 2
_NS = 16
_LANES = 16
_NW = _NC * _NS

_CHUNK = 128
_RBLK = 131072


@functools.lru_cache(maxsize=None)
def _build_format(d: int, table_rows: int):
    nj = pl.cdiv(table_rows, _RBLK)
    ni = d // 8

    def fmt_kernel(in_ref, pass_ref, flat_ref):
        x = in_ref[...]
        pass_ref[...] = x
        flat_ref[...] = x.reshape(8, _RBLK // 128, 128).transpose(1, 0, 2)[None]

    return pl.pallas_call(
        fmt_kernel,
        grid=(ni, nj),
        in_specs=[pl.BlockSpec((8, _RBLK), lambda i, j: (i, j))],
        out_specs=[
            pl.BlockSpec((8, _RBLK), lambda i, j: (i, j)),
            pl.BlockSpec((1, _RBLK // 128, 8, 128), lambda i, j: (i * nj + j, 0, 0, 0)),
        ],
        out_shape=[
            jax.ShapeDtypeStruct((d, table_rows), jnp.float32),
            jax.ShapeDtypeStruct((ni * nj, _RBLK // 128, 8, 128), jnp.float32),
        ],
        compiler_params=pltpu.CompilerParams(
            dimension_semantics=("parallel", "arbitrary"),
        ),
    )


@functools.lru_cache(maxsize=None)
def _build_gather(n_rows: int, d: int, table_rows: int):
    assert d % _NW == 0
    cols_per_w = d // _NW
    assert n_rows % _CHUNK == 0
    n_ch = n_rows // _CHUNK
    vregs_per_chunk = _CHUNK // _LANES
    nj = pl.cdiv(table_rows, _RBLK)

    mesh = plsc.VectorSubcoreMesh(core_axis_name="c", subcore_axis_name="s")

    @functools.partial(
        pl.kernel,
        mesh=mesh,
        out_type=jax.ShapeDtypeStruct((d, n_ch, _CHUNK), jnp.float32),
        scratch_types=[
            pltpu.VMEM((n_ch, _CHUNK), jnp.int32),
            pltpu.VMEM((n_ch, _CHUNK), jnp.float32),
            pltpu.SemaphoreType.DMA,
        ],
    )
    def gather_kernel(flat_hbm, idxT_hbm, outT_hbm, idx_v, val_v, sem):
        wid = lax.axis_index("s") * _NC + lax.axis_index("c")

        def do_col(k, carry):
            j = wid * cols_per_w + k

            pltpu.sync_copy(idxT_hbm.at[j], idx_v)

            base = ((j // 8) * nj) * (8 * _RBLK) + (j % 8) * 128

            def conv(c, carry2):
                for v in range(vregs_per_chunk):
                    sl = pl.ds(v * _LANES, _LANES)
                    x = idx_v[c, sl]
                    f = base + ((x >> 17) << 20)
                    f = f + (((x >> 7) & 1023) << 10)
                    idx_v[c, sl] = f + (x & 127)
                return carry2

            lax.fori_loop(0, n_ch, conv, 0)

            def fire(c, carry2):
                pltpu.async_copy(flat_hbm.at[idx_v.at[c]], val_v.at[c], sem)
                return carry2

            lax.fori_loop(0, n_ch, fire, 0)
            pltpu.make_async_copy(outT_hbm.at[j], val_v, sem).wait()

            pltpu.sync_copy(val_v, outT_hbm.at[j])
            return carry

        lax.fori_loop(0, cols_per_w, do_col, 0)

    return gather_kernel


def kernel(input, index, _):
    table_rows, d = input.shape
    n_rows = index.shape[0]

    passT, flat3 = _build_format(d, table_rows)(input.T)
    flat = flat3.reshape(flat3.shape[0] * 8 * _RBLK)

    idxT3 = index.T.reshape(d, n_rows // _CHUNK, _CHUNK)
    gathered = _build_gather(n_rows, d, table_rows)(flat, idxT3)
    return (passT.T, index, gathered.reshape(d, n_rows).T)


# speedup vs baseline: 2.6989x; 1.0266x over previous
import functools

import jax
import jax.numpy as jnp
from jax import lax
from jax.experimental import pallas as pl
from jax.experimental.pallas import tpu as pltpu
from jax.experimental.pallas import tpu_sc as plsc

_NC = 2
_NS = 16
_LANES = 16
_NW = _NC * _NS

_CHUNK = 128
_RBLK_LOG = 18
_RBLK = 1 << _RBLK_LOG


@functools.lru_cache(maxsize=None)
def _build_format(d: int, table_rows: int):
    nj = pl.cdiv(table_rows, _RBLK)
    ni = d // 8

    def fmt_kernel(in_ref, pass_ref, flat_ref):
        x = in_ref[...]
        pass_ref[...] = x
        flat_ref[...] = x.reshape(8, _RBLK // 128, 128).transpose(1, 0, 2)[None]

    return pl.pallas_call(
        fmt_kernel,
        grid=(ni, nj),
        in_specs=[pl.BlockSpec((8, _RBLK), lambda i, j: (i, j))],
        out_specs=[
            pl.BlockSpec((8, _RBLK), lambda i, j: (i, j)),
            pl.BlockSpec((1, _RBLK // 128, 8, 128), lambda i, j: (i * nj + j, 0, 0, 0)),
        ],
        out_shape=[
            jax.ShapeDtypeStruct((d, table_rows), jnp.float32),
            jax.ShapeDtypeStruct((ni * nj, _RBLK // 128, 8, 128), jnp.float32),
        ],
        compiler_params=pltpu.CompilerParams(
            dimension_semantics=("parallel", "arbitrary"),
        ),
    )


@functools.lru_cache(maxsize=None)
def _build_gather(n_rows: int, d: int, table_rows: int):
    assert d % _NW == 0
    cols_per_w = d // _NW
    assert n_rows % _CHUNK == 0
    n_ch = n_rows // _CHUNK
    vregs_per_chunk = _CHUNK // _LANES
    nj = pl.cdiv(table_rows, _RBLK)

    mesh = plsc.VectorSubcoreMesh(core_axis_name="c", subcore_axis_name="s")

    @functools.partial(
        pl.kernel,
        mesh=mesh,
        out_type=jax.ShapeDtypeStruct((d, n_ch, _CHUNK), jnp.float32),
        scratch_types=[
            pltpu.VMEM((n_ch, _CHUNK), jnp.int32),
            pltpu.VMEM((n_ch, _CHUNK), jnp.float32),
            pltpu.SemaphoreType.DMA,
        ],
    )
    def gather_kernel(flat_hbm, idxT_hbm, outT_hbm, idx_v, val_v, sem):
        wid = lax.axis_index("s") * _NC + lax.axis_index("c")

        def do_col(k, carry):
            j = wid * cols_per_w + k

            pltpu.sync_copy(idxT_hbm.at[j], idx_v)

            base = ((j // 8) * nj) * (8 * _RBLK) + (j % 8) * 128
            tmask = _RBLK // 128 - 1

            def conv_fire(c, carry2):
                for v in range(vregs_per_chunk):
                    sl = pl.ds(v * _LANES, _LANES)
                    x = idx_v[c, sl]
                    f = base + ((x >> _RBLK_LOG) << (_RBLK_LOG + 3))
                    f = f + (((x >> 7) & tmask) << 10)
                    idx_v[c, sl] = f + (x & 127)
                pltpu.async_copy(flat_hbm.at[idx_v.at[c]], val_v.at[c], sem)
                return carry2

            lax.fori_loop(0, n_ch, conv_fire, 0)
            pltpu.make_async_copy(outT_hbm.at[j], val_v, sem).wait()

            pltpu.sync_copy(val_v, outT_hbm.at[j])
            return carry

        lax.fori_loop(0, cols_per_w, do_col, 0)

    return gather_kernel


def kernel(input, index, _):
    table_rows, d = input.shape
    n_rows = index.shape[0]

    passT, flat3 = _build_format(d, table_rows)(input.T)
    flat = flat3.reshape(flat3.shape[0] * 8 * _RBLK)

    idxT3 = index.T.reshape(d, n_rows // _CHUNK, _CHUNK)
    gathered = _build_gather(n_rows, d, table_rows)(flat, idxT3)
    return (passT.T, index, gathered.reshape(d, n_rows).T)
